# baseline (speedup 1.0000x reference)
import math

import jax
import jax.numpy as jnp
from jax import lax
from jax.experimental import pallas as pl
from jax.experimental.pallas import tpu as pltpu
from jax.experimental.pallas import tpu_sc as plsc

N = 10000
E = 320000
F = 128
HID = 256

NC = 2
NS = 16
HALF = N // 2
ROWS = HALF + 120
TPT = ROWS // NS
W0 = 144
CS0 = F
CBL = F + 1
CS1 = F + 2
CHUNK = E // NS
SUB = 2000
SUBG = 2048
SGP = SUBG - SUB
SUBP = SUB + 272
NSUB = CHUNK // SUB
BLK = 64
NQ = 4
QN = N // NQ
QROWS = QN + 60
TPQ = QROWS // NS

_f32 = jnp.float32
_i32 = jnp.int32


def _edge_body(x0_h, x1_h, x2_h, v_h, a_h, src_h, dst_h, prm_h,
               o0_h, o1_h, bw_h,
               xs0b, xs1b, xs2b, xd0b, xd1b, xd2b, vsrc,
               pv, src_s, dst_s, c0_s, c1_s, bw_s, kj,
               gix, dix, kblk, rows_v, gix2, dix2, kblk2, rows_v2,
               out0_v, out1_v,
               o0_acc, o1_acc, gsem, gsem2, xsem):
    c = lax.axis_index("c")
    s = lax.axis_index("s")
    i16 = jnp.arange(16, dtype=_i32)
    z16i = jnp.zeros((16,), _i32)
    z16f = jnp.zeros((16,), _f32)
    d1v = z16i + 1
    d2v = z16i + 2

    pltpu.sync_copy(prm_h, pv)
    prm = pv[pl.ds(0, 16)]
    inv_h = prm[0]
    sigma = prm[1]

    for k in range(SGP // 16):
        src_s[pl.ds(SUB + 16 * k, 16)] = z16i
    c0_s[pl.ds(SUB, 16)] = z16f
    c1_s[pl.ds(SUB, 16)] = z16f
    bw_s[pl.ds(SUB, 16)] = z16f

    def _zero_row(r, _):
        for k in range(W0 // 16):
            out0_v[r, pl.ds(16 * k, 16)] = z16f
        for k in range(F // 16):
            out1_v[r, pl.ds(16 * k, 16)] = z16f
        return 0

    r0 = s * TPQ
    for p in range(2):
        q = 2 * p + c
        lo = q * QN
        for k in range(SGP // 16):
            dst_s[pl.ds(SUB + 16 * k, 16)] = z16i + lo
        lax.fori_loop(0, BLK, _zero_row, 0)

        pltpu.sync_copy(out0_v, o0_acc.at[pl.ds(r0, BLK)])
        pltpu.sync_copy(out0_v, o0_acc.at[pl.ds(r0 + BLK, BLK)])
        pltpu.sync_copy(out0_v.at[pl.ds(0, TPQ - 2 * BLK)],
                        o0_acc.at[pl.ds(r0 + 2 * BLK, TPQ - 2 * BLK)])
        pltpu.sync_copy(out1_v, o1_acc.at[pl.ds(r0, BLK)])
        pltpu.sync_copy(out1_v, o1_acc.at[pl.ds(r0 + BLK, BLK)])
        pltpu.sync_copy(out1_v.at[pl.ds(0, TPQ - 2 * BLK)],
                        o1_acc.at[pl.ds(r0 + 2 * BLK, TPQ - 2 * BLK)])
        plsc.subcore_barrier()

        def _sub_chunk(sc, _):
            base = s * CHUNK + sc * SUB
            pltpu.sync_copy(src_h.at[pl.ds(base, SUB)],
                            src_s.at[pl.ds(0, SUB)])
            pltpu.sync_copy(dst_h.at[pl.ds(base, SUB)],
                            dst_s.at[pl.ds(0, SUB)])

            descs = []
            for k in range(SUBG // 128):
                ix = pl.ds(k * 128, 128)
                for (tab, buf, idx) in ((x0_h, xs0b, src_s), (x1_h, xs1b, src_s),
                                        (x2_h, xs2b, src_s), (x0_h, xd0b, dst_s),
                                        (x1_h, xd1b, dst_s), (x2_h, xd2b, dst_s),
                                        (v_h, vsrc, src_s)):
                    descs.append(pltpu.async_copy(
                        tab.at[idx.at[ix]], buf.at[ix], xsem))
            for d in descs:
                d.wait()

            def _pass_a(b, kcount):
                off = b * 16
                srcv = src_s[pl.ds(off, 16)]
                dstv = dst_s[pl.ds(off, 16)]
                dx0 = xd0b[pl.ds(off, 16)] - xs0b[pl.ds(off, 16)]
                dx1 = xd1b[pl.ds(off, 16)] - xs1b[pl.ds(off, 16)]
                dx2 = xd2b[pl.ds(off, 16)] - xs2b[pl.ds(off, 16)]
                r2 = dx0 * dx0 + dx1 * dx1 + dx2 * dx2 + 1e-12
                iy = jnp.int32(0x5F3759DF) - lax.shift_right_arithmetic(
                    plsc.bitcast(r2, _i32), 1)
                y = plsc.bitcast(iy, _f32)
                y = y * (1.5 - 0.5 * r2 * y * y)
                y = y * (1.5 - 0.5 * r2 * y * y)
                y = y * (1.5 - 0.5 * r2 * y * y)
                r = r2 * y
                qq = r * inv_h
                q2 = qq * qq
                one_q = 1.0 - qq
                w_in = 6.0 * (q2 * qq - q2) + 1.0
                w_out = 2.0 * one_q * one_q * one_q
                inb = qq <= 1.0
                w = jnp.where(inb,
                              sigma * jnp.where(qq <= 0.5, w_in, w_out), 0.0)
                d_in = 6.0 * (3.0 * q2 - 2.0 * qq)
                d_out = -6.0 * one_q * one_q
                dwdq = jnp.where(inb,
                                 sigma * jnp.where(qq <= 0.5, d_in, d_out),
                                 0.0)
                vs = vsrc[pl.ds(off, 16)]
                bw = vs * w
                gfac = dwdq * inv_h * (1.0 / (r + 1e-8)) * vs
                c0v = gfac * dx0
                c1v = gfac * dx1
                c0_s[pl.ds(off, 16)] = c0v
                c1_s[pl.ds(off, 16)] = c1v
                bw_s[pl.ds(off, 16)] = bw
                dl = dstv - lo
                keep = (dl >= 0) & (dl < QN) & inb
                plsc.store_compressed(kj.at[pl.ds(kcount, 16)], i16 + off,
                                      mask=keep)
                cnt = plsc.all_reduce_population_count(keep)
                return kcount + cnt[0]

            kcount = lax.fori_loop(0, SUB // 16, _pass_a, 0)
            for k in range(16):
                kj[pl.ds(kcount + 16 * k, 16)] = z16i + SUB

            @pl.when((lax.rem(sc, 2) == c) & (p == 0))
            def _():
                pltpu.sync_copy(bw_s.at[pl.ds(0, SUB)],
                                bw_h.at[pl.ds(base, SUB)])

            nmb = (kcount + BLK - 1) // BLK
            nmb1 = jnp.maximum(nmb, 1)
            np2 = (nmb1 + 1) // 2

            gbufs = ((gix, dix, kblk, rows_v, gsem),
                     (gix2, dix2, kblk2, rows_v2, gsem2))

            def _build(mb, par):
                gx, dx, kb, rv_, gs = gbufs[par]
                for g in range(BLK // 16):
                    off = mb * BLK + g * 16
                    kjv = kj[pl.ds(off, 16)]
                    srcv = plsc.load_gather(src_s, [kjv])
                    dlv = plsc.load_gather(dst_s, [kjv]) - lo
                    gx[0, pl.ds(g * 16, 16)] = srcv
                    dx[0, pl.ds(g * 16, 16)] = dlv
                    kb[0, pl.ds(g * 16, 16)] = kjv
                pltpu.async_copy(a_h.at[gx.at[0]], rv_, gs)

            def _phase(mb, par):
                gx, dx, kb, rv_, gs = gbufs[par]
                pltpu.make_async_copy(a_h.at[gx.at[0]], rv_, gs).wait()
                _build(mb + 1, 1 - par)

                def _row(r, _):
                    j = kb[0, pl.ds(r, 16)][0]
                    c0r = c0_s[pl.ds(j, 16)][0]
                    c1r = c1_s[pl.ds(j, 16)][0]
                    bwr = bw_s[pl.ds(j, 16)][0]
                    bm0 = jnp.where(rv_[r, pl.ds(0, 16)][3] > 0.1,
                                    bwr, 0.0)
                    for k in range(F // 16):
                        rvk = rv_[r, pl.ds(16 * k, 16)]
                        out0_v[r, pl.ds(16 * k, 16)] = rvk * c0r
                        out1_v[r, pl.ds(16 * k, 16)] = rvk * c1r
                    ev = jnp.where(i16 == 0, c0r,
                                   jnp.where(i16 == 1, bm0,
                                             jnp.where(i16 == 2, c1r, 0.0)))
                    out0_v[r, pl.ds(F, 16)] = ev
                    return 0

                lax.fori_loop(0, BLK, _row, 0)
                pltpu.sync_copy(out0_v, o0_acc.at[dx.at[0]], add=True)
                pltpu.sync_copy(out1_v, o1_acc.at[dx.at[0]], add=True)

            _build(0, 0)

            def _pair(i2, _):
                _phase(i2 * 2, 0)
                _phase(i2 * 2 + 1, 1)
                return 0

            lax.fori_loop(0, np2, _pair, 0)
            gx, dx, kb, rv_, gs = gbufs[0]
            pltpu.make_async_copy(a_h.at[gx.at[0]], rv_, gs).wait()
            return 0

        lax.fori_loop(0, NSUB, _sub_chunk, 0)
        plsc.subcore_barrier()

        ob = q * QROWS + r0
        pltpu.sync_copy(o0_acc.at[pl.ds(r0, TPQ)], o0_h.at[pl.ds(ob, TPQ)])
        pltpu.sync_copy(o1_acc.at[pl.ds(r0, TPQ)], o1_h.at[pl.ds(ob, TPQ)])
        plsc.subcore_barrier()


def _make_edge_kernel():
    mesh = plsc.VectorSubcoreMesh(core_axis_name="c", subcore_axis_name="s",
                                  num_cores=NC, num_subcores=NS)
    return pl.kernel(
        _edge_body,
        out_type=(
            jax.ShapeDtypeStruct((NQ * QROWS, W0), _f32),
            jax.ShapeDtypeStruct((NQ * QROWS, F), _f32),
            jax.ShapeDtypeStruct((E,), _f32),
        ),
        mesh=mesh,
        compiler_params=pltpu.CompilerParams(use_tc_tiling_on_sc=False,
                                             needs_layout_passes=False),
        scratch_types=[
            pltpu.VMEM((SUBG,), _f32),
            pltpu.VMEM((SUBG,), _f32),
            pltpu.VMEM((SUBG,), _f32),
            pltpu.VMEM((SUBG,), _f32),
            pltpu.VMEM((SUBG,), _f32),
            pltpu.VMEM((SUBG,), _f32),
            pltpu.VMEM((SUBG,), _f32),
            pltpu.VMEM((16,), _f32),
            pltpu.VMEM((SUBP,), _i32),
            pltpu.VMEM((SUBP,), _i32),
            pltpu.VMEM((SUBP,), _f32),
            pltpu.VMEM((SUBP,), _f32),
            pltpu.VMEM((SUBP,), _f32),
            pltpu.VMEM((SUBP,), _i32),
            pltpu.VMEM((1, BLK), _i32),
            pltpu.VMEM((1, BLK), _i32),
            pltpu.VMEM((1, BLK + 16), _i32),
            pltpu.VMEM((BLK, F), _f32),
            pltpu.VMEM((1, BLK), _i32),
            pltpu.VMEM((1, BLK), _i32),
            pltpu.VMEM((1, BLK + 16), _i32),
            pltpu.VMEM((BLK, F), _f32),
            pltpu.VMEM((BLK, W0), _f32),
            pltpu.VMEM((BLK, F), _f32),
            pltpu.VMEM_SHARED((QROWS, W0), _f32),
            pltpu.VMEM_SHARED((QROWS, F), _f32),
            pltpu.SemaphoreType.DMA,
            pltpu.SemaphoreType.DMA,
            pltpu.SemaphoreType.DMA,
        ],
    )


BCHUNK = CHUNK + 96
NBB = BCHUNK // 128


def _blur_body(m1_h, bw_h, src_h, dst_h, na_h, out_h,
               m1v, srcb, dstb, bwb, valb, dix, accv, narows, acc_sp):
    c = lax.axis_index("c")
    s = lax.axis_index("s")
    lo = c * HALF
    z16i = jnp.zeros((16,), _i32)
    z16f = jnp.zeros((16,), _f32)

    pltpu.sync_copy(m1_h, m1v)
    pltpu.sync_copy(src_h.at[pl.ds(s * CHUNK, CHUNK)],
                    srcb.at[pl.ds(0, CHUNK)])
    pltpu.sync_copy(dst_h.at[pl.ds(s * CHUNK, CHUNK)],
                    dstb.at[pl.ds(0, CHUNK)])
    pltpu.sync_copy(bw_h.at[pl.ds(s * CHUNK, CHUNK)], bwb.at[pl.ds(0, CHUNK)])
    for k in range(6):
        srcb[pl.ds(CHUNK + 16 * k, 16)] = z16i
        dstb[pl.ds(CHUNK + 16 * k, 16)] = z16i
        bwb[pl.ds(CHUNK + 16 * k, 16)] = z16f

    def _zv(i, _):
        valb[pl.ds(16 * i, 16)] = z16f
        return 0
    lax.fori_loop(0, 128 // 16, _zv, 0)
    r0 = s * TPT
    pltpu.sync_copy(valb, acc_sp.at[pl.ds(r0, 128)])
    pltpu.sync_copy(valb, acc_sp.at[pl.ds(r0 + 128, 128)])
    pltpu.sync_copy(valb.at[pl.ds(0, TPT - 256)],
                    acc_sp.at[pl.ds(r0 + 256, TPT - 256)])
    plsc.subcore_barrier()

    def _blk(mb, _):
        for g in range(128 // 16):
            off = mb * 128 + g * 16
            srcv = srcb[pl.ds(off, 16)]
            dstv = dstb[pl.ds(off, 16)]
            bwv = bwb[pl.ds(off, 16)]
            mv = plsc.load_gather(m1v, [srcv])
            dl = dstv - lo
            inh = (dl >= 0) & (dl < HALF)
            valb[pl.ds(g * 16, 16)] = jnp.where(inh, bwv * mv, 0.0)
            dix[0, pl.ds(g * 16, 16)] = jnp.where(inh, dl, 0)
        pltpu.sync_copy(valb, acc_sp.at[dix.at[0]], add=True)
        return 0

    lax.fori_loop(0, NBB, _blk, 0)
    plsc.subcore_barrier()

    pltpu.sync_copy(acc_sp.at[pl.ds(r0, TPT)], accv.at[pl.ds(0, TPT)])
    gb = c * HALF + r0
    lastn = HALF - (NS - 1) * TPT
    cnt = jnp.where(s == NS - 1, lastn, TPT)

    @pl.when(s < NS - 1)
    def _():
        pltpu.sync_copy(na_h.at[pl.ds(gb, TPT)], narows.at[pl.ds(0, TPT)])

    @pl.when(s == NS - 1)
    def _():
        pltpu.sync_copy(na_h.at[pl.ds(gb, lastn)], narows.at[pl.ds(0, lastn)])

    def _fin(r, _):
        lf = jnp.where(accv[pl.ds(r, 16)][0] > 0.1, 1.0, 0.0)
        for k in range(F // 16):
            narows[r, pl.ds(16 * k, 16)] = narows[r, pl.ds(16 * k, 16)] * lf
        return 0

    lax.fori_loop(0, cnt, _fin, 0)

    @pl.when(s < NS - 1)
    def _():
        pltpu.sync_copy(narows.at[pl.ds(0, TPT)], out_h.at[pl.ds(gb, TPT)])

    @pl.when(s == NS - 1)
    def _():
        pltpu.sync_copy(narows.at[pl.ds(0, lastn)], out_h.at[pl.ds(gb, lastn)])


def _make_blur_kernel():
    mesh = plsc.VectorSubcoreMesh(core_axis_name="c", subcore_axis_name="s",
                                  num_cores=NC, num_subcores=NS)
    return pl.kernel(
        _blur_body,
        out_type=jax.ShapeDtypeStruct((N, F), _f32),
        mesh=mesh,
        compiler_params=pltpu.CompilerParams(use_tc_tiling_on_sc=False,
                                             needs_layout_passes=False),
        scratch_types=[
            pltpu.VMEM((N,), _f32),
            pltpu.VMEM((BCHUNK,), _i32),
            pltpu.VMEM((BCHUNK,), _i32),
            pltpu.VMEM((BCHUNK,), _f32),
            pltpu.VMEM((128,), _f32),
            pltpu.VMEM((1, 128), _i32),
            pltpu.VMEM((TPT + 16,), _f32),
            pltpu.VMEM((TPT, F), _f32),
            pltpu.VMEM_SHARED((ROWS,), _f32),
        ],
    )


MB = 1000


def _mlp_body(a_ref, p0_ref, p1_ref, s0_ref, s1_ref, b0_ref, um_ref,
              w1t_ref, b1_ref, w2t_ref, b2_ref, na_ref, m1_ref):
    a = a_ref[...]
    s0 = s0_ref[...]
    s1 = s1_ref[...]
    y = jnp.concatenate([a, p0_ref[...] - a * s0, p1_ref[...] - a * s1],
                        axis=1)
    hid = jnp.maximum(
        jnp.dot(y, w1t_ref[...], preferred_element_type=_f32) + b1_ref[...],
        0.0)
    da = jnp.dot(hid, w2t_ref[...], preferred_element_type=_f32) + b2_ref[...]
    gate = jax.nn.sigmoid(da[:, :F])
    delta = jnp.tanh(da[:, F:2 * F])
    mult = jax.nn.sigmoid(da[:, 2 * F:2 * F + 1])
    um = um_ref[...]
    na = a * gate + delta * mult
    na = um * na + (1.0 - um) * a
    m1_ref[...] = jnp.where(na[:, 3:4] > 0.1, 1.0, 0.0)
    prev = jnp.where(b0_ref[...] > 0.1, 1.0, 0.0)
    na_ref[...] = na * prev


def _mlp(a, p0, p1, s0, s1, b0, um, w1t, b1, w2t, b2):
    nb = N // MB
    row = lambda i: (i, 0)
    full = lambda i: (0, 0)
    return pl.pallas_call(
        _mlp_body,
        grid=(nb,),
        in_specs=[
            pl.BlockSpec((MB, F), row),
            pl.BlockSpec((MB, F), row),
            pl.BlockSpec((MB, F), row),
            pl.BlockSpec((MB, 1), row),
            pl.BlockSpec((MB, 1), row),
            pl.BlockSpec((MB, 1), row),
            pl.BlockSpec((MB, 1), row),
            pl.BlockSpec((3 * F, HID), full),
            pl.BlockSpec((1, HID), full),
            pl.BlockSpec((HID, 3 * F), full),
            pl.BlockSpec((1, 3 * F), full),
        ],
        out_specs=[
            pl.BlockSpec((MB, F), row),
            pl.BlockSpec((MB, 1), row),
        ],
        out_shape=[
            jax.ShapeDtypeStruct((N, F), _f32),
            jax.ShapeDtypeStruct((N, 1), _f32),
        ],
    )(a, p0, p1, s0, s1, b0, um, w1t, b1, w2t, b2)


def kernel(x, v, A, h, edge_index, W1, b1, W2, b2):
    src = edge_index[0]
    dst = edge_index[1]
    hf = h.astype(_f32)
    inv_h = 1.0 / hf
    sigma = 8.0 / (math.pi * hf ** 3)
    params = jnp.zeros((16,), _f32).at[0].set(inv_h).at[1].set(sigma)

    o0, o1, bw = _make_edge_kernel()(x[:, 0], x[:, 1], x[:, 2], v, A,
                                     src, dst, params)
    o0 = o0.reshape(NQ, QROWS, W0)
    o1 = o1.reshape(NQ, QROWS, F)
    p0 = o0[:, :QN, :F].reshape(N, F)
    s0 = o0[:, :QN, CS0].reshape(N, 1)
    b0 = o0[:, :QN, CBL].reshape(N, 1)
    s1 = o0[:, :QN, CS1].reshape(N, 1)
    p1 = o1[:, :QN, :].reshape(N, F)

    um = (jax.random.uniform(jax.random.key(42), (N,)) <= 0.5)
    um = um.astype(_f32)[:, None]
    w1t = W1.T
    w2p = jnp.zeros((3 * F, HID), _f32).at[:2 * F + 1].set(W2)
    b2p = jnp.zeros((3 * F,), _f32).at[:2 * F + 1].set(b2)

    na1, m1 = _mlp(A, p0, p1, s0, s1, b0, um, w1t, b1.reshape(1, HID),
                   w2p.T, b2p.reshape(1, 3 * F))

    na = _make_blur_kernel()(m1.reshape(N), bw, src, dst, na1)
    return (x, na)


# speedup vs baseline: 2.0304x; 2.0304x over previous
import math

import jax
import jax.numpy as jnp
from jax import lax
from jax.experimental import pallas as pl
from jax.experimental.pallas import tpu as pltpu
from jax.experimental.pallas import tpu_sc as plsc

N = 10000
E = 320000
F = 128
HID = 256

NC = 2
NS = 16
HALF = N // 2
ROWS = HALF + 120
TPT = ROWS // NS
W0 = 144
CS0 = F
CBL = F + 1
CS1 = F + 2
CHUNK = E // NS
SUB = 2000
SUBG = 2048
SGP = SUBG - SUB
SUBP = SUB + 144
NSUB = CHUNK // SUB
BLK = 64
NQ = 4
QN = N // NQ
QROWS = QN + 60
TPQ = QROWS // NS

_f32 = jnp.float32
_i32 = jnp.int32


def _edge_body(x0_h, x1_h, x2_h, v_h, a_h, src_h, dst_h, prm_h,
               o0_h, o1_h, bw_h,
               xs0b, xs1b, xs2b, xd0b, xd1b, xd2b, vsrc,
               pv, src_s, dst_s, c0_s, c1_s, bw_s, kj,
               gix, dix, kblk, rows_v, out0_v, out1_v,
               o0_acc, o1_acc, gsem):
    c = lax.axis_index("c")
    s = lax.axis_index("s")
    i16 = jnp.arange(16, dtype=_i32)
    z16i = jnp.zeros((16,), _i32)
    z16f = jnp.zeros((16,), _f32)
    d1v = z16i + 1
    d2v = z16i + 2

    pltpu.sync_copy(prm_h, pv)
    prm = pv[pl.ds(0, 16)]
    inv_h = prm[0]
    sigma = prm[1]

    for k in range(SGP // 16):
        src_s[pl.ds(SUB + 16 * k, 16)] = z16i
    c0_s[pl.ds(SUB, 16)] = z16f
    c1_s[pl.ds(SUB, 16)] = z16f
    bw_s[pl.ds(SUB, 16)] = z16f

    def _zero_row(r, _):
        for k in range(W0 // 16):
            out0_v[r, pl.ds(16 * k, 16)] = z16f
        for k in range(F // 16):
            out1_v[r, pl.ds(16 * k, 16)] = z16f
        return 0

    r0 = s * TPQ
    for p in range(2):
        q = 2 * p + c
        lo = q * QN
        for k in range(SGP // 16):
            dst_s[pl.ds(SUB + 16 * k, 16)] = z16i + lo
        lax.fori_loop(0, BLK, _zero_row, 0)

        pltpu.sync_copy(out0_v, o0_acc.at[pl.ds(r0, BLK)])
        pltpu.sync_copy(out0_v, o0_acc.at[pl.ds(r0 + BLK, BLK)])
        pltpu.sync_copy(out0_v.at[pl.ds(0, TPQ - 2 * BLK)],
                        o0_acc.at[pl.ds(r0 + 2 * BLK, TPQ - 2 * BLK)])
        pltpu.sync_copy(out1_v, o1_acc.at[pl.ds(r0, BLK)])
        pltpu.sync_copy(out1_v, o1_acc.at[pl.ds(r0 + BLK, BLK)])
        pltpu.sync_copy(out1_v.at[pl.ds(0, TPQ - 2 * BLK)],
                        o1_acc.at[pl.ds(r0 + 2 * BLK, TPQ - 2 * BLK)])
        plsc.subcore_barrier()

        def _sub_chunk(sc, _):
            base = s * CHUNK + sc * SUB
            pltpu.sync_copy(src_h.at[pl.ds(base, SUB)],
                            src_s.at[pl.ds(0, SUB)])
            pltpu.sync_copy(dst_h.at[pl.ds(base, SUB)],
                            dst_s.at[pl.ds(0, SUB)])

            descs = []
            for k in range(SUBG // 128):
                ix = pl.ds(k * 128, 128)
                for (tab, buf, idx) in ((x0_h, xs0b, src_s), (x1_h, xs1b, src_s),
                                        (x2_h, xs2b, src_s), (x0_h, xd0b, dst_s),
                                        (x1_h, xd1b, dst_s), (x2_h, xd2b, dst_s),
                                        (v_h, vsrc, src_s)):
                    descs.append(pltpu.async_copy(
                        tab.at[idx.at[ix]], buf.at[ix], gsem))
            for d in descs:
                d.wait()

            def _pass_a(b, kcount):
                off = b * 16
                srcv = src_s[pl.ds(off, 16)]
                dstv = dst_s[pl.ds(off, 16)]
                dx0 = xd0b[pl.ds(off, 16)] - xs0b[pl.ds(off, 16)]
                dx1 = xd1b[pl.ds(off, 16)] - xs1b[pl.ds(off, 16)]
                dx2 = xd2b[pl.ds(off, 16)] - xs2b[pl.ds(off, 16)]
                r2 = dx0 * dx0 + dx1 * dx1 + dx2 * dx2 + 1e-12
                iy = jnp.int32(0x5F3759DF) - lax.shift_right_arithmetic(
                    plsc.bitcast(r2, _i32), 1)
                y = plsc.bitcast(iy, _f32)
                y = y * (1.5 - 0.5 * r2 * y * y)
                y = y * (1.5 - 0.5 * r2 * y * y)
                y = y * (1.5 - 0.5 * r2 * y * y)
                r = r2 * y
                qq = r * inv_h
                q2 = qq * qq
                one_q = 1.0 - qq
                w_in = 6.0 * (q2 * qq - q2) + 1.0
                w_out = 2.0 * one_q * one_q * one_q
                inb = qq <= 1.0
                w = jnp.where(inb,
                              sigma * jnp.where(qq <= 0.5, w_in, w_out), 0.0)
                d_in = 6.0 * (3.0 * q2 - 2.0 * qq)
                d_out = -6.0 * one_q * one_q
                dwdq = jnp.where(inb,
                                 sigma * jnp.where(qq <= 0.5, d_in, d_out),
                                 0.0)
                vs = vsrc[pl.ds(off, 16)]
                bw = vs * w
                gfac = dwdq * inv_h * (1.0 / (r + 1e-8)) * vs
                c0v = gfac * dx0
                c1v = gfac * dx1
                c0_s[pl.ds(off, 16)] = c0v
                c1_s[pl.ds(off, 16)] = c1v
                bw_s[pl.ds(off, 16)] = bw
                dl = dstv - lo
                keep = (dl >= 0) & (dl < QN) & inb
                plsc.store_compressed(kj.at[pl.ds(kcount, 16)], i16 + off,
                                      mask=keep)
                cnt = plsc.all_reduce_population_count(keep)
                return kcount + cnt[0]

            kcount = lax.fori_loop(0, SUB // 16, _pass_a, 0)
            for k in range(BLK // 16):
                kj[pl.ds(kcount + 16 * k, 16)] = z16i + SUB

            @pl.when((lax.rem(sc, 2) == c) & (p == 0))
            def _():
                pltpu.sync_copy(bw_s.at[pl.ds(0, SUB)],
                                bw_h.at[pl.ds(base, SUB)])

            nmb = (kcount + BLK - 1) // BLK

            def _pass_b(mb, _):
                for g in range(BLK // 16):
                    off = mb * BLK + g * 16
                    kjv = kj[pl.ds(off, 16)]
                    srcv = plsc.load_gather(src_s, [kjv])
                    dlv = plsc.load_gather(dst_s, [kjv]) - lo
                    gix[0, pl.ds(g * 16, 16)] = srcv
                    dix[0, pl.ds(g * 16, 16)] = dlv
                    kblk[0, pl.ds(g * 16, 16)] = kjv
                pltpu.async_copy(a_h.at[gix.at[0]], rows_v, gsem).wait()

                def _row(r, _):
                    j = kblk[0, pl.ds(r, 16)][0]
                    c0r = c0_s[pl.ds(j, 16)][0]
                    c1r = c1_s[pl.ds(j, 16)][0]
                    bwr = bw_s[pl.ds(j, 16)][0]
                    bm0 = jnp.where(rows_v[r, pl.ds(0, 16)][3] > 0.1,
                                    bwr, 0.0)
                    for k in range(F // 16):
                        rv = rows_v[r, pl.ds(16 * k, 16)]
                        out0_v[r, pl.ds(16 * k, 16)] = rv * c0r
                        out1_v[r, pl.ds(16 * k, 16)] = rv * c1r
                    ev = jnp.where(i16 == 0, c0r,
                                   jnp.where(i16 == 1, bm0,
                                             jnp.where(i16 == 2, c1r, 0.0)))
                    out0_v[r, pl.ds(F, 16)] = ev
                    return 0

                lax.fori_loop(0, BLK, _row, 0)
                pltpu.sync_copy(out0_v, o0_acc.at[dix.at[0]], add=True)
                pltpu.sync_copy(out1_v, o1_acc.at[dix.at[0]], add=True)
                return 0

            lax.fori_loop(0, nmb, _pass_b, 0)
            return 0

        lax.fori_loop(0, NSUB, _sub_chunk, 0)
        plsc.subcore_barrier()

        ob = q * QROWS + r0
        pltpu.sync_copy(o0_acc.at[pl.ds(r0, TPQ)], o0_h.at[pl.ds(ob, TPQ)])
        pltpu.sync_copy(o1_acc.at[pl.ds(r0, TPQ)], o1_h.at[pl.ds(ob, TPQ)])
        plsc.subcore_barrier()


def _make_edge_kernel():
    mesh = plsc.VectorSubcoreMesh(core_axis_name="c", subcore_axis_name="s",
                                  num_cores=NC, num_subcores=NS)
    return pl.kernel(
        _edge_body,
        out_type=(
            jax.ShapeDtypeStruct((NQ * QROWS, W0), _f32),
            jax.ShapeDtypeStruct((NQ * QROWS, F), _f32),
            jax.ShapeDtypeStruct((E,), _f32),
        ),
        mesh=mesh,
        compiler_params=pltpu.CompilerParams(use_tc_tiling_on_sc=False,
                                             needs_layout_passes=False),
        scratch_types=[
            pltpu.VMEM((SUBG,), _f32),
            pltpu.VMEM((SUBG,), _f32),
            pltpu.VMEM((SUBG,), _f32),
            pltpu.VMEM((SUBG,), _f32),
            pltpu.VMEM((SUBG,), _f32),
            pltpu.VMEM((SUBG,), _f32),
            pltpu.VMEM((SUBG,), _f32),
            pltpu.VMEM((16,), _f32),
            pltpu.VMEM((SUBP,), _i32),
            pltpu.VMEM((SUBP,), _i32),
            pltpu.VMEM((SUBP,), _f32),
            pltpu.VMEM((SUBP,), _f32),
            pltpu.VMEM((SUBP,), _f32),
            pltpu.VMEM((SUBP,), _i32),
            pltpu.VMEM((1, BLK), _i32),
            pltpu.VMEM((1, BLK), _i32),
            pltpu.VMEM((1, BLK + 16), _i32),
            pltpu.VMEM((BLK, F), _f32),
            pltpu.VMEM((BLK, W0), _f32),
            pltpu.VMEM((BLK, F), _f32),
            pltpu.VMEM_SHARED((QROWS, W0), _f32),
            pltpu.VMEM_SHARED((QROWS, F), _f32),
            pltpu.SemaphoreType.DMA,
        ],
    )


BCHUNK = CHUNK + 96
NBB = BCHUNK // 128


def _blur_body(m1_h, bw_h, src_h, dst_h, na_h, out_h,
               m1v, srcb, dstb, bwb, valb, dix, accv, narows, acc_sp):
    c = lax.axis_index("c")
    s = lax.axis_index("s")
    lo = c * HALF
    z16i = jnp.zeros((16,), _i32)
    z16f = jnp.zeros((16,), _f32)

    pltpu.sync_copy(m1_h, m1v)
    pltpu.sync_copy(src_h.at[pl.ds(s * CHUNK, CHUNK)],
                    srcb.at[pl.ds(0, CHUNK)])
    pltpu.sync_copy(dst_h.at[pl.ds(s * CHUNK, CHUNK)],
                    dstb.at[pl.ds(0, CHUNK)])
    pltpu.sync_copy(bw_h.at[pl.ds(s * CHUNK, CHUNK)], bwb.at[pl.ds(0, CHUNK)])
    for k in range(6):
        srcb[pl.ds(CHUNK + 16 * k, 16)] = z16i
        dstb[pl.ds(CHUNK + 16 * k, 16)] = z16i
        bwb[pl.ds(CHUNK + 16 * k, 16)] = z16f

    def _zv(i, _):
        valb[pl.ds(16 * i, 16)] = z16f
        return 0
    lax.fori_loop(0, 128 // 16, _zv, 0)
    r0 = s * TPT
    pltpu.sync_copy(valb, acc_sp.at[pl.ds(r0, 128)])
    pltpu.sync_copy(valb, acc_sp.at[pl.ds(r0 + 128, 128)])
    pltpu.sync_copy(valb.at[pl.ds(0, TPT - 256)],
                    acc_sp.at[pl.ds(r0 + 256, TPT - 256)])
    plsc.subcore_barrier()

    def _blk(mb, _):
        for g in range(128 // 16):
            off = mb * 128 + g * 16
            srcv = srcb[pl.ds(off, 16)]
            dstv = dstb[pl.ds(off, 16)]
            bwv = bwb[pl.ds(off, 16)]
            mv = plsc.load_gather(m1v, [srcv])
            dl = dstv - lo
            inh = (dl >= 0) & (dl < HALF)
            valb[pl.ds(g * 16, 16)] = jnp.where(inh, bwv * mv, 0.0)
            dix[0, pl.ds(g * 16, 16)] = jnp.where(inh, dl, 0)
        pltpu.sync_copy(valb, acc_sp.at[dix.at[0]], add=True)
        return 0

    lax.fori_loop(0, NBB, _blk, 0)
    plsc.subcore_barrier()

    pltpu.sync_copy(acc_sp.at[pl.ds(r0, TPT)], accv.at[pl.ds(0, TPT)])
    gb = c * HALF + r0
    lastn = HALF - (NS - 1) * TPT
    cnt = jnp.where(s == NS - 1, lastn, TPT)

    @pl.when(s < NS - 1)
    def _():
        pltpu.sync_copy(na_h.at[pl.ds(gb, TPT)], narows.at[pl.ds(0, TPT)])

    @pl.when(s == NS - 1)
    def _():
        pltpu.sync_copy(na_h.at[pl.ds(gb, lastn)], narows.at[pl.ds(0, lastn)])

    def _fin(r, _):
        lf = jnp.where(accv[pl.ds(r, 16)][0] > 0.1, 1.0, 0.0)
        for k in range(F // 16):
            narows[r, pl.ds(16 * k, 16)] = narows[r, pl.ds(16 * k, 16)] * lf
        return 0

    lax.fori_loop(0, cnt, _fin, 0)

    @pl.when(s < NS - 1)
    def _():
        pltpu.sync_copy(narows.at[pl.ds(0, TPT)], out_h.at[pl.ds(gb, TPT)])

    @pl.when(s == NS - 1)
    def _():
        pltpu.sync_copy(narows.at[pl.ds(0, lastn)], out_h.at[pl.ds(gb, lastn)])


def _make_blur_kernel():
    mesh = plsc.VectorSubcoreMesh(core_axis_name="c", subcore_axis_name="s",
                                  num_cores=NC, num_subcores=NS)
    return pl.kernel(
        _blur_body,
        out_type=jax.ShapeDtypeStruct((N, F), _f32),
        mesh=mesh,
        compiler_params=pltpu.CompilerParams(use_tc_tiling_on_sc=False,
                                             needs_layout_passes=False),
        scratch_types=[
            pltpu.VMEM((N,), _f32),
            pltpu.VMEM((BCHUNK,), _i32),
            pltpu.VMEM((BCHUNK,), _i32),
            pltpu.VMEM((BCHUNK,), _f32),
            pltpu.VMEM((128,), _f32),
            pltpu.VMEM((1, 128), _i32),
            pltpu.VMEM((TPT + 16,), _f32),
            pltpu.VMEM((TPT, F), _f32),
            pltpu.VMEM_SHARED((ROWS,), _f32),
        ],
    )


MB = 1000


def _mlp_body(a_ref, p0_ref, p1_ref, s0_ref, s1_ref, b0_ref, um_ref,
              w1t_ref, b1_ref, w2t_ref, b2_ref, na_ref, m1_ref):
    a = a_ref[...]
    s0 = s0_ref[...]
    s1 = s1_ref[...]
    y = jnp.concatenate([a, p0_ref[...] - a * s0, p1_ref[...] - a * s1],
                        axis=1)
    hid = jnp.maximum(
        jnp.dot(y, w1t_ref[...], preferred_element_type=_f32) + b1_ref[...],
        0.0)
    da = jnp.dot(hid, w2t_ref[...], preferred_element_type=_f32) + b2_ref[...]
    gate = jax.nn.sigmoid(da[:, :F])
    delta = jnp.tanh(da[:, F:2 * F])
    mult = jax.nn.sigmoid(da[:, 2 * F:2 * F + 1])
    um = um_ref[...]
    na = a * gate + delta * mult
    na = um * na + (1.0 - um) * a
    m1_ref[...] = jnp.where(na[:, 3:4] > 0.1, 1.0, 0.0)
    prev = jnp.where(b0_ref[...] > 0.1, 1.0, 0.0)
    na_ref[...] = na * prev


def _mlp(a, p0, p1, s0, s1, b0, um, w1t, b1, w2t, b2):
    nb = N // MB
    row = lambda i: (i, 0)
    full = lambda i: (0, 0)
    return pl.pallas_call(
        _mlp_body,
        grid=(nb,),
        in_specs=[
            pl.BlockSpec((MB, F), row),
            pl.BlockSpec((MB, F), row),
            pl.BlockSpec((MB, F), row),
            pl.BlockSpec((MB, 1), row),
            pl.BlockSpec((MB, 1), row),
            pl.BlockSpec((MB, 1), row),
            pl.BlockSpec((MB, 1), row),
            pl.BlockSpec((3 * F, HID), full),
            pl.BlockSpec((1, HID), full),
            pl.BlockSpec((HID, 3 * F), full),
            pl.BlockSpec((1, 3 * F), full),
        ],
        out_specs=[
            pl.BlockSpec((MB, F), row),
            pl.BlockSpec((MB, 1), row),
        ],
        out_shape=[
            jax.ShapeDtypeStruct((N, F), _f32),
            jax.ShapeDtypeStruct((N, 1), _f32),
        ],
    )(a, p0, p1, s0, s1, b0, um, w1t, b1, w2t, b2)


def kernel(x, v, A, h, edge_index, W1, b1, W2, b2):
    src = edge_index[0]
    dst = edge_index[1]
    hf = h.astype(_f32)
    inv_h = 1.0 / hf
    sigma = 8.0 / (math.pi * hf ** 3)
    params = jnp.zeros((16,), _f32).at[0].set(inv_h).at[1].set(sigma)

    o0, o1, bw = _make_edge_kernel()(x[:, 0], x[:, 1], x[:, 2], v, A,
                                     src, dst, params)
    o0 = o0.reshape(NQ, QROWS, W0)
    o1 = o1.reshape(NQ, QROWS, F)
    p0 = o0[:, :QN, :F].reshape(N, F)
    s0 = o0[:, :QN, CS0].reshape(N, 1)
    b0 = o0[:, :QN, CBL].reshape(N, 1)
    s1 = o0[:, :QN, CS1].reshape(N, 1)
    p1 = o1[:, :QN, :].reshape(N, F)

    um = (jax.random.uniform(jax.random.key(42), (N,)) <= 0.5)
    um = um.astype(_f32)[:, None]
    w1t = W1.T
    w2p = jnp.zeros((3 * F, HID), _f32).at[:2 * F + 1].set(W2)
    b2p = jnp.zeros((3 * F,), _f32).at[:2 * F + 1].set(b2)

    na1, m1 = _mlp(A, p0, p1, s0, s1, b0, um, w1t, b1.reshape(1, HID),
                   w2p.T, b2p.reshape(1, 3 * F))

    na = _make_blur_kernel()(m1.reshape(N), bw, src, dst, na1)
    return (x, na)
